# baseline (speedup 1.0000x reference)
import functools
import math

import jax
import jax.numpy as jnp
from jax import lax
from jax.experimental import pallas as pl
from jax.experimental.pallas import tpu as pltpu
from jax.experimental.pallas import tpu_sc as plsc

D_MODEL = 64
SCALE = math.sqrt(D_MODEL)
NUM_CORES = 2
NUM_SUBCORES = 16
NUM_WORKERS = NUM_CORES * NUM_SUBCORES
LANES = 16
VPR = D_MODEL // LANES
NBUF = 4
ROW_UNROLL = 2


def _embed_sc(x, table):
    n_rows, row_len = x.shape
    rows_per_w = n_rows // NUM_WORKERS
    mesh = plsc.VectorSubcoreMesh(core_axis_name="c", subcore_axis_name="s")

    @functools.partial(
        pl.kernel,
        mesh=mesh,
        out_type=jax.ShapeDtypeStruct((n_rows, row_len, D_MODEL), jnp.float32),
        scratch_types=[
            pltpu.VMEM((rows_per_w, row_len), jnp.int32),
        ]
        + [pltpu.VMEM((row_len, D_MODEL), jnp.float32) for _ in range(2 * NBUF)]
        + [pltpu.SemaphoreType.DMA for _ in range(2 * NBUF)],
        compiler_params=pltpu.CompilerParams(use_tc_tiling_on_sc=False),
    )
    def k(x_hbm, table_hbm, out_hbm, idx_v, *bufs_and_sems):
        ins = bufs_and_sems[0:NBUF]
        outs = bufs_and_sems[NBUF:2 * NBUF]
        gsems = bufs_and_sems[2 * NBUF:3 * NBUF]
        wsems = bufs_and_sems[3 * NBUF:4 * NBUF]

        wid = lax.axis_index("s") * NUM_CORES + lax.axis_index("c")
        base = wid * rows_per_w

        pltpu.sync_copy(x_hbm.at[pl.ds(base, rows_per_w), :], idx_v)

        def start_gather(ci, b):
            pltpu.async_copy(table_hbm.at[idx_v.at[ci]], ins[b], gsems[b])

        def scale(b):
            src = ins[b]
            dst = outs[b]

            def body(r, c):
                rr = r * ROW_UNROLL
                for u in range(ROW_UNROLL):
                    for j in range(VPR):
                        sl = pl.ds(j * LANES, LANES)
                        dst[rr + u, sl] = src[rr + u, sl] * SCALE
                return c

            lax.fori_loop(0, row_len // ROW_UNROLL, body, 0)

        for b in range(NBUF):
            start_gather(b, b)

        def ring_body(r, carry):
            for b in range(NBUF):
                ci = r * NBUF + b
                pltpu.make_async_copy(
                    table_hbm.at[idx_v.at[0]], ins[b], gsems[b]).wait()
                @pl.when(ci >= NBUF)
                def _():
                    pltpu.make_async_copy(
                        outs[b], out_hbm.at[base], wsems[b]).wait()
                scale(b)
                @pl.when(ci + NBUF < rows_per_w)
                def _():
                    start_gather(ci + NBUF, b)
                pltpu.async_copy(outs[b], out_hbm.at[base + ci], wsems[b])
            return carry

        lax.fori_loop(0, rows_per_w // NBUF, ring_body, 0)

        for b in range(NBUF):
            pltpu.make_async_copy(
                outs[b], out_hbm.at[base], wsems[b]).wait()

    return k(x, table)


def kernel(x, table):
    return _embed_sc(x.astype(jnp.int32), table)
